# baseline (speedup 1.0000x reference)
import functools

import jax
import jax.numpy as jnp
from jax import lax
from jax.experimental import pallas as pl
from jax.experimental.pallas import tpu as pltpu
from jax.experimental.pallas import tpu_sc as plsc

N = 10000
E = 320000
D = 128
H = 8
HID = 8
C = 16

NC = 2
NS = 16
NW = NC * NS

NPAD = 10240
RPW = NPAD // NS
CH = 208
CHUNKS = 50
EPW = CH * CHUNKS
EPAD = EPW * NW

_mesh = plsc.VectorSubcoreMesh(core_axis_name="c", subcore_axis_name="s")


def _iota16():
    return lax.iota(jnp.int32, 16)


def _mp1_body(s_hbm, d_hbm, src3, dst3, zacc,
              acc_out, srcs, dsts, sA, sB, dA, dB, cA, cB,
              acc_sh, gsA, gsB, gdA, gdB, scA, scB):
    c_idx = lax.axis_index("c")
    s_idx = lax.axis_index("s")
    wid = s_idx * NC + c_idx
    r0 = s_idx * RPW
    pltpu.sync_copy(zacc.at[pl.ds(r0, RPW)], acc_sh.at[pl.ds(r0, RPW)])
    pltpu.sync_copy(src3.at[wid], srcs)
    pltpu.sync_copy(dst3.at[wid], dsts)
    plsc.subcore_barrier()

    lanes = _iota16()

    def gstart(j, sbuf, dbuf, gs, gd):
        pltpu.async_copy(s_hbm.at[srcs.at[j]], sbuf, gs)
        pltpu.async_copy(d_hbm.at[dsts.at[j]], dbuf, gd)

    def gwait(sbuf, dbuf, gs, gd):
        pltpu.make_async_copy(s_hbm.at[srcs.at[0]], sbuf, gs).wait()
        pltpu.make_async_copy(d_hbm.at[dsts.at[0]], dbuf, gd).wait()

    def swait(cbuf, sc):
        pltpu.make_async_copy(zacc.at[pl.ds(0, CH)], cbuf, sc).wait()

    def compute(s_rows, d_rows, c_v):
        def blk(i2, carry):
            row = i2 * 16 + lanes
            exs = []
            for k in range(H):
                ck = jnp.full((16,), k, jnp.int32)
                a = plsc.load_gather(s_rows, [row, ck + 64])
                b = plsc.load_gather(d_rows, [row, ck])
                al = a + b
                al = jnp.maximum(al, 0.2 * al)
                e = jnp.exp(al)
                plsc.store_scatter(c_v, [row, ck + 64], e)
                exs.append(e)
            for h in range(H):
                for q in range(HID):
                    col = jnp.full((16,), h * HID + q, jnp.int32)
                    xwv = plsc.load_gather(s_rows, [row, col])
                    plsc.store_scatter(c_v, [row, col], xwv * exs[h])
            return carry

        lax.fori_loop(0, CH // 16, blk, 0)

    gstart(0, sA, dA, gsA, gdA)
    gstart(1, sB, dB, gsB, gdB)

    def pipe(i, carry):
        jA = 2 * i
        jB = 2 * i + 1
        gwait(sA, dA, gsA, gdA)

        @pl.when(i > 0)
        def _():
            swait(cA, scA)

        compute(sA, dA, cA)
        pltpu.async_copy(cA, acc_sh.at[dsts.at[jA]], scA, add=True)

        @pl.when(i < CHUNKS // 2 - 1)
        def _():
            gstart(jA + 2, sA, dA, gsA, gdA)

        gwait(sB, dB, gsB, gdB)

        @pl.when(i > 0)
        def _():
            swait(cB, scB)

        compute(sB, dB, cB)
        pltpu.async_copy(cB, acc_sh.at[dsts.at[jB]], scB, add=True)

        @pl.when(i < CHUNKS // 2 - 1)
        def _():
            gstart(jB + 2, sB, dB, gsB, gdB)

        return carry

    lax.fori_loop(0, CHUNKS // 2, pipe, 0)
    swait(cA, scA)
    swait(cB, scB)
    plsc.subcore_barrier()
    pltpu.sync_copy(acc_sh.at[pl.ds(r0, RPW)], acc_out.at[c_idx, pl.ds(r0, RPW)])


def _mp1(s_tab, d_tab, src3, dst3, zacc):
    kfn = pl.kernel(
        _mp1_body,
        mesh=_mesh,
        compiler_params=pltpu.CompilerParams(
            needs_layout_passes=False, use_tc_tiling_on_sc=False),
        out_type=jax.ShapeDtypeStruct((NC, NPAD, 72), jnp.float32),
        scratch_types=[
            pltpu.VMEM((CHUNKS, CH), jnp.int32),
            pltpu.VMEM((CHUNKS, CH), jnp.int32),
            pltpu.VMEM((CH, 72), jnp.float32),
            pltpu.VMEM((CH, 72), jnp.float32),
            pltpu.VMEM((CH, H), jnp.float32),
            pltpu.VMEM((CH, H), jnp.float32),
            pltpu.VMEM((CH, 72), jnp.float32),
            pltpu.VMEM((CH, 72), jnp.float32),
            pltpu.VMEM_SHARED((NPAD, 72), jnp.float32),
            pltpu.SemaphoreType.DMA,
            pltpu.SemaphoreType.DMA,
            pltpu.SemaphoreType.DMA,
            pltpu.SemaphoreType.DMA,
            pltpu.SemaphoreType.DMA,
            pltpu.SemaphoreType.DMA,
        ],
    )
    return kfn(s_tab, d_tab, src3, dst3, zacc)


W2ACC = 24


def _mp2_body(s_hbm, as_hbm, ad_hbm, src3, dst3, zacc,
              acc_out, srcs, dsts, sA, sB, asA, asB, adA, adB, cA, cB,
              acc_sh, gsA, gsB, gaA, gaB, gdA, gdB, scA, scB):
    c_idx = lax.axis_index("c")
    s_idx = lax.axis_index("s")
    wid = s_idx * NC + c_idx
    r0 = s_idx * RPW
    pltpu.sync_copy(zacc.at[pl.ds(r0, RPW)], acc_sh.at[pl.ds(r0, RPW)])
    pltpu.sync_copy(src3.at[wid], srcs)
    pltpu.sync_copy(dst3.at[wid], dsts)
    plsc.subcore_barrier()

    lanes = _iota16()
    c16 = jnp.full((16,), C, jnp.int32)

    def gstart(j, sbuf, abuf, dbuf, gs, ga, gd):
        pltpu.async_copy(s_hbm.at[srcs.at[j]], sbuf, gs)
        pltpu.async_copy(as_hbm.at[srcs.at[j]], abuf, ga)
        pltpu.async_copy(ad_hbm.at[dsts.at[j]], dbuf, gd)

    def gwait(sbuf, abuf, dbuf, gs, ga, gd):
        pltpu.make_async_copy(s_hbm.at[srcs.at[0]], sbuf, gs).wait()
        pltpu.make_async_copy(as_hbm.at[srcs.at[0]], abuf, ga).wait()
        pltpu.make_async_copy(ad_hbm.at[dsts.at[0]], dbuf, gd).wait()

    def swait(cbuf, sc):
        pltpu.make_async_copy(zacc.at[pl.ds(0, CH)], cbuf, sc).wait()

    def compute(s_rows, as_v, ad_v, c_v):
        def blk(i2, carry):
            row = i2 * 16 + lanes
            sl = pl.ds(i2 * 16, 16)
            al = as_v[sl] + ad_v[sl]
            al = jnp.maximum(al, 0.2 * al)
            e = jnp.exp(al)
            plsc.store_scatter(c_v, [row, c16], e)
            for q in range(C):
                col = jnp.full((16,), q, jnp.int32)
                xwv = plsc.load_gather(s_rows, [row, col])
                plsc.store_scatter(c_v, [row, col], xwv * e)
            return carry

        lax.fori_loop(0, CH // 16, blk, 0)

    gstart(0, sA, asA, adA, gsA, gaA, gdA)
    gstart(1, sB, asB, adB, gsB, gaB, gdB)

    def pipe(i, carry):
        jA = 2 * i
        jB = 2 * i + 1
        gwait(sA, asA, adA, gsA, gaA, gdA)

        @pl.when(i > 0)
        def _():
            swait(cA, scA)

        compute(sA, asA, adA, cA)
        pltpu.async_copy(cA, acc_sh.at[dsts.at[jA]], scA, add=True)

        @pl.when(i < CHUNKS // 2 - 1)
        def _():
            gstart(jA + 2, sA, asA, adA, gsA, gaA, gdA)

        gwait(sB, asB, adB, gsB, gaB, gdB)

        @pl.when(i > 0)
        def _():
            swait(cB, scB)

        compute(sB, asB, adB, cB)
        pltpu.async_copy(cB, acc_sh.at[dsts.at[jB]], scB, add=True)

        @pl.when(i < CHUNKS // 2 - 1)
        def _():
            gstart(jB + 2, sB, asB, adB, gsB, gaB, gdB)

        return carry

    lax.fori_loop(0, CHUNKS // 2, pipe, 0)
    swait(cA, scA)
    swait(cB, scB)
    plsc.subcore_barrier()
    pltpu.sync_copy(acc_sh.at[pl.ds(r0, RPW)], acc_out.at[c_idx, pl.ds(r0, RPW)])


def _mp2(s_tab, as_n, ad_n, src3, dst3, zacc):
    kfn = pl.kernel(
        _mp2_body,
        mesh=_mesh,
        compiler_params=pltpu.CompilerParams(
            needs_layout_passes=False, use_tc_tiling_on_sc=False),
        out_type=jax.ShapeDtypeStruct((NC, NPAD, W2ACC), jnp.float32),
        scratch_types=[
            pltpu.VMEM((CHUNKS, CH), jnp.int32),
            pltpu.VMEM((CHUNKS, CH), jnp.int32),
            pltpu.VMEM((CH, C), jnp.float32),
            pltpu.VMEM((CH, C), jnp.float32),
            pltpu.VMEM((CH,), jnp.float32),
            pltpu.VMEM((CH,), jnp.float32),
            pltpu.VMEM((CH,), jnp.float32),
            pltpu.VMEM((CH,), jnp.float32),
            pltpu.VMEM((CH, W2ACC), jnp.float32),
            pltpu.VMEM((CH, W2ACC), jnp.float32),
            pltpu.VMEM_SHARED((NPAD, W2ACC), jnp.float32),
            pltpu.SemaphoreType.DMA,
            pltpu.SemaphoreType.DMA,
            pltpu.SemaphoreType.DMA,
            pltpu.SemaphoreType.DMA,
            pltpu.SemaphoreType.DMA,
            pltpu.SemaphoreType.DMA,
            pltpu.SemaphoreType.DMA,
            pltpu.SemaphoreType.DMA,
        ],
    )
    return kfn(s_tab, as_n, ad_n, src3, dst3, zacc)


BI = 1024
GRID = NPAD // BI


def _tabs1_body(x_ref, w1_ref, a1s_ref, a1d_ref, s_ref, ad_ref):
    xw = jnp.dot(x_ref[...], w1_ref[...], preferred_element_type=jnp.float32)
    s_ref[:, :H * HID] = xw
    s_ref[:, H * HID:] = jnp.dot(xw, a1s_ref[...],
                                 preferred_element_type=jnp.float32)
    ad_ref[...] = jnp.dot(xw, a1d_ref[...], preferred_element_type=jnp.float32)


def _tc_tabs1(xp, W1, A1s, A1d):
    return pl.pallas_call(
        _tabs1_body,
        grid=(GRID,),
        in_specs=[
            pl.BlockSpec((BI, D), lambda i: (i, 0)),
            pl.BlockSpec((D, H * HID), lambda i: (0, 0)),
            pl.BlockSpec((H * HID, H), lambda i: (0, 0)),
            pl.BlockSpec((H * HID, H), lambda i: (0, 0)),
        ],
        out_specs=[
            pl.BlockSpec((BI, 72), lambda i: (i, 0)),
            pl.BlockSpec((BI, H), lambda i: (i, 0)),
        ],
        out_shape=[
            jax.ShapeDtypeStruct((NPAD, 72), jnp.float32),
            jax.ShapeDtypeStruct((NPAD, H), jnp.float32),
        ],
    )(xp, W1, A1s, A1d)


def _tabs2_body(acc_ref, rexp_ref, b1_ref, w2_ref, a2s_ref, a2d_ref,
                xw2_ref, as2_ref, ad2_ref):
    acct = acc_ref[0] + acc_ref[1]
    numt = acct[:, :H * HID]
    dent = acct[:, H * HID:]
    denx = jnp.dot(dent, rexp_ref[...],
                   preferred_element_type=jnp.float32)
    h = jax.nn.relu(numt / denx + b1_ref[...])
    xw2 = jnp.dot(h, w2_ref[...], preferred_element_type=jnp.float32)
    xw2_ref[...] = xw2
    as2_ref[...] = jnp.dot(xw2, a2s_ref[...], preferred_element_type=jnp.float32)
    ad2_ref[...] = jnp.dot(xw2, a2d_ref[...], preferred_element_type=jnp.float32)


def _tc_tabs2(acc1, Rexp, b1r, W2, a2sT, a2dT):
    return pl.pallas_call(
        _tabs2_body,
        grid=(GRID,),
        in_specs=[
            pl.BlockSpec((NC, BI, 72), lambda i: (0, i, 0)),
            pl.BlockSpec((H, H * HID), lambda i: (0, 0)),
            pl.BlockSpec((1, H * HID), lambda i: (0, 0)),
            pl.BlockSpec((H * HID, C), lambda i: (0, 0)),
            pl.BlockSpec((C, 1), lambda i: (0, 0)),
            pl.BlockSpec((C, 1), lambda i: (0, 0)),
        ],
        out_specs=[
            pl.BlockSpec((BI, C), lambda i: (i, 0)),
            pl.BlockSpec((BI, 1), lambda i: (i, 0)),
            pl.BlockSpec((BI, 1), lambda i: (i, 0)),
        ],
        out_shape=[
            jax.ShapeDtypeStruct((NPAD, C), jnp.float32),
            jax.ShapeDtypeStruct((NPAD, 1), jnp.float32),
            jax.ShapeDtypeStruct((NPAD, 1), jnp.float32),
        ],
    )(acc1, Rexp, b1r, W2, a2sT, a2dT)


def _x1key_body(acc_ref, b2_ref, pw_ref, pb_ref,
                x1_ref, key_ref, gx_ref):
    i = pl.program_id(0)
    acct = acc_ref[0] + acc_ref[1]
    numt = acct[:, :C]
    dent = acct[:, C:C + 1]
    x1 = numt / dent + b2_ref[...]
    x1_ref[...] = x1
    g = jnp.dot(x1, pw_ref[...], preferred_element_type=jnp.float32) \
        + pb_ref[...]
    rows = jax.lax.broadcasted_iota(jnp.int32, (BI, 1), 0) + i * BI
    key_ref[...] = jnp.where(rows < N, g, jnp.inf)
    rows16 = jax.lax.broadcasted_iota(jnp.int32, (BI, C), 0) + i * BI
    gx_ref[...] = jnp.where(rows16 < N, g * x1, 0.0)


def _tc_x1key(acc2, b2r, pwT, pbr):
    return pl.pallas_call(
        _x1key_body,
        grid=(GRID,),
        in_specs=[
            pl.BlockSpec((NC, BI, W2ACC), lambda i: (0, i, 0)),
            pl.BlockSpec((1, C), lambda i: (0, 0)),
            pl.BlockSpec((C, 1), lambda i: (0, 0)),
            pl.BlockSpec((1, 1), lambda i: (0, 0)),
        ],
        out_specs=[
            pl.BlockSpec((BI, C), lambda i: (i, 0)),
            pl.BlockSpec((BI, 1), lambda i: (i, 0)),
            pl.BlockSpec((BI, C), lambda i: (i, 0)),
        ],
        out_shape=[
            jax.ShapeDtypeStruct((NPAD, C), jnp.float32),
            jax.ShapeDtypeStruct((NPAD, 1), jnp.float32),
            jax.ShapeDtypeStruct((NPAD, C), jnp.float32),
        ],
    )(acc2, b2r, pwT, pbr)


BJ = 2048


def _rank_body(ki_ref, kr_ref, rank_ref, acc_ref):
    i = pl.program_id(0)
    ki = ki_ref[...]
    jrel = (jax.lax.broadcasted_iota(jnp.int32, (BI, BJ), 1)
            - jax.lax.broadcasted_iota(jnp.int32, (BI, BJ), 0))
    acc_ref[...] = jnp.zeros((BI, BJ), jnp.float32)

    def jchunk(jc, carry):
        kj = kr_ref[0, pl.ds(jc * BJ, BJ)].reshape(1, BJ)
        lt = kj < ki
        acc_ref[...] += jnp.where(lt, 1.0, 0.0)

        @pl.when(jc * BJ < (i + 1) * BI)
        def _():
            eqb = (kj == ki) & (jrel < i * BI - jc * BJ)
            acc_ref[...] += jnp.where(eqb, 1.0, 0.0)

        return carry

    lax.fori_loop(0, NPAD // BJ, jchunk, 0)
    rank_ref[...] = jnp.sum(acc_ref[...], axis=1, keepdims=True).astype(jnp.int32)


def _tc_rank(key_col, key_row):
    return pl.pallas_call(
        _rank_body,
        grid=(GRID,),
        in_specs=[
            pl.BlockSpec((BI, 1), lambda i: (i, 0)),
            pl.BlockSpec((1, NPAD), lambda i: (0, 0)),
        ],
        out_specs=pl.BlockSpec((BI, 1), lambda i: (i, 0)),
        out_shape=jax.ShapeDtypeStruct((NPAD, 1), jnp.int32),
        scratch_shapes=[pltpu.VMEM((BI, BJ), jnp.float32)],
    )(key_col, key_row)


def _conv_body(sin_ref, wc1_ref, wc2_ref, b1_ref, b2_ref, l2_ref,
               z_ref, h1p_ref):
    acc = jnp.zeros((NPAD, C), jnp.float32)
    for k in range(5):
        acc += jnp.dot(sin_ref[pl.ds(k, NPAD), :], wc1_ref[pl.ds(k * C, C), :],
                       preferred_element_type=jnp.float32)
    rows = jax.lax.broadcasted_iota(jnp.int32, (NPAD, C), 0)
    h1 = jnp.where(rows < N, jax.nn.relu(acc + b1_ref[...]), 0.0)
    h1p_ref[pl.ds(0, 2), :] = jnp.zeros((2, C), jnp.float32)
    h1p_ref[pl.ds(2, NPAD), :] = h1
    h1p_ref[pl.ds(NPAD + 2, 2), :] = jnp.zeros((2, C), jnp.float32)
    acc2 = jnp.zeros((NPAD, C), jnp.float32)
    for k in range(5):
        acc2 += jnp.dot(h1p_ref[pl.ds(k, NPAD), :], wc2_ref[pl.ds(k * C, C), :],
                        preferred_element_type=jnp.float32)
    y = acc2 + b2_ref[...]
    z_ref[...] = jnp.dot(y, l2_ref[...], preferred_element_type=jnp.float32)


def _tc_conv(sin_pad, Wc1, Wc2, c1br, c2br, L2):
    return pl.pallas_call(
        _conv_body,
        in_specs=[
            pl.BlockSpec((NPAD + 4, C), lambda: (0, 0)),
            pl.BlockSpec((5 * C, C), lambda: (0, 0)),
            pl.BlockSpec((5 * C, C), lambda: (0, 0)),
            pl.BlockSpec((1, C), lambda: (0, 0)),
            pl.BlockSpec((1, C), lambda: (0, 0)),
            pl.BlockSpec((C, C), lambda: (0, 0)),
        ],
        out_specs=pl.BlockSpec((NPAD, C), lambda: (0, 0)),
        out_shape=jax.ShapeDtypeStruct((NPAD, C), jnp.float32),
        scratch_shapes=[pltpu.VMEM((NPAD + 4, C), jnp.float32)],
    )(sin_pad, Wc1, Wc2, c1br, c2br, L2)


def _final_body(x1_ref, zu_ref, l1_ref, b_ref, out_ref):
    o = jnp.dot(x1_ref[...], l1_ref[...], preferred_element_type=jnp.float32) \
        + zu_ref[...] + b_ref[...]
    m = jnp.max(o, axis=1, keepdims=True)
    s = o - m
    out_ref[...] = s - jnp.log(jnp.sum(jnp.exp(s), axis=1, keepdims=True))


def _tc_final(x1, zu, L1, br):
    return pl.pallas_call(
        _final_body,
        grid=(GRID,),
        in_specs=[
            pl.BlockSpec((BI, C), lambda i: (i, 0)),
            pl.BlockSpec((BI, C), lambda i: (i, 0)),
            pl.BlockSpec((C, C), lambda i: (0, 0)),
            pl.BlockSpec((1, C), lambda i: (0, 0)),
        ],
        out_specs=pl.BlockSpec((BI, C), lambda i: (i, 0)),
        out_shape=jax.ShapeDtypeStruct((NPAD, C), jnp.float32),
    )(x1, zu, L1, br)


RPW2 = NPAD // NW


def _scat_body(gx_hbm, rank_hbm, out_hbm, idx_v, rows_v, sem1):
    c_idx = lax.axis_index("c")
    s_idx = lax.axis_index("s")
    r0 = (s_idx * NC + c_idx) * RPW2
    pltpu.sync_copy(rank_hbm.at[pl.ds(r0, RPW2)], idx_v)
    pltpu.sync_copy(gx_hbm.at[pl.ds(r0, RPW2)], rows_v)
    pltpu.async_copy(rows_v, out_hbm.at[idx_v], sem1).wait()


def _sc_scatter_rows(gx, rank):
    kfn = pl.kernel(
        _scat_body,
        mesh=_mesh,
        compiler_params=pltpu.CompilerParams(
            needs_layout_passes=False, use_tc_tiling_on_sc=False),
        out_type=jax.ShapeDtypeStruct((NPAD, C), jnp.float32),
        scratch_types=[
            pltpu.VMEM((RPW2,), jnp.int32),
            pltpu.VMEM((RPW2, C), jnp.float32),
            pltpu.SemaphoreType.DMA,
        ],
    )
    return kfn(gx, rank)


def _gath_body(z_hbm, rank_hbm, out_hbm, idx_v, rows_v, sem1):
    c_idx = lax.axis_index("c")
    s_idx = lax.axis_index("s")
    r0 = (s_idx * NC + c_idx) * RPW2
    pltpu.sync_copy(rank_hbm.at[pl.ds(r0, RPW2)], idx_v)
    pltpu.async_copy(z_hbm.at[idx_v], rows_v, sem1).wait()
    pltpu.sync_copy(rows_v, out_hbm.at[pl.ds(r0, RPW2)])


def _sc_gather_rows(z, rank):
    kfn = pl.kernel(
        _gath_body,
        mesh=_mesh,
        compiler_params=pltpu.CompilerParams(
            needs_layout_passes=False, use_tc_tiling_on_sc=False),
        out_type=jax.ShapeDtypeStruct((NPAD, C), jnp.float32),
        scratch_types=[
            pltpu.VMEM((RPW2,), jnp.int32),
            pltpu.VMEM((RPW2, C), jnp.float32),
            pltpu.SemaphoreType.DMA,
        ],
    )
    return kfn(z, rank)


def kernel(x, edge_index, W1, a1_src, a1_dst, b1, W2, a2_src, a2_dst, b2,
           proj_w, proj_b, c1_w, c1_b, c2_w, c2_b, lin_w, lin_b):
    loop = jnp.arange(N, dtype=jnp.int32)
    src = jnp.concatenate([edge_index[0].astype(jnp.int32), loop])
    dst = jnp.concatenate([edge_index[1].astype(jnp.int32), loop])
    npad_e = EPAD - (E + N)
    pad_idx = 10000 + (jnp.arange(npad_e, dtype=jnp.int32) % (NPAD - N))
    srcp = jnp.concatenate([src, pad_idx])
    dstp = jnp.concatenate([dst, pad_idx])

    eyeH = jnp.eye(H, dtype=jnp.float32)
    A1s = (a1_src[:, :, None] * eyeH[:, None, :]).reshape(H * HID, H)
    A1d = (a1_dst[:, :, None] * eyeH[:, None, :]).reshape(H * HID, H)
    Rexp = jnp.kron(eyeH, jnp.ones((1, HID), jnp.float32))
    b1r = b1.reshape(1, H * HID)
    a2sT = a2_src.T
    a2dT = a2_dst.T
    b2r = b2.reshape(1, C)
    pwT = proj_w.T
    pbr = proj_b.reshape(1, 1)
    Wc1 = jnp.transpose(c1_w, (2, 1, 0)).reshape(5 * C, C)
    Wc2 = jnp.transpose(c2_w, (2, 1, 0)).reshape(5 * C, C)
    c1br = c1_b.reshape(1, C)
    c2br = c2_b.reshape(1, C)
    L1 = lin_w[:, :C].T
    L2 = lin_w[:, C:].T
    br = lin_b.reshape(1, C)

    src3 = srcp.reshape(NW, CHUNKS, CH)
    dst3 = dstp.reshape(NW, CHUNKS, CH)

    xp = jnp.pad(x, ((0, NPAD - N), (0, 0)))
    s_tab, ad1 = _tc_tabs1(xp, W1, A1s, A1d)
    zacc1 = jnp.zeros((NPAD, 72), jnp.float32)
    acc1 = _mp1(s_tab, ad1, src3, dst3, zacc1)

    xw2, as2, ad2 = _tc_tabs2(acc1, Rexp, b1r, W2, a2sT, a2dT)
    zacc2 = jnp.zeros((NPAD, W2ACC), jnp.float32)
    acc2 = _mp2(xw2, as2.reshape(NPAD), ad2.reshape(NPAD),
                src3, dst3, zacc2)

    x1p, key, gx = _tc_x1key(acc2, b2r, pwT, pbr)
    rank = _tc_rank(key, key.reshape(1, NPAD)).reshape(NPAD)
    sorted_in = _sc_scatter_rows(gx, rank)

    sin_pad = jnp.pad(sorted_in, ((2, 2), (0, 0)))
    z = _tc_conv(sin_pad, Wc1, Wc2, c1br, c2br, L2)
    zu = _sc_gather_rows(z, rank)
    outp = _tc_final(x1p, zu, L1, br)
    return outp[:N]


# speedup vs baseline: 1.0022x; 1.0022x over previous
import functools

import jax
import jax.numpy as jnp
from jax import lax
from jax.experimental import pallas as pl
from jax.experimental.pallas import tpu as pltpu
from jax.experimental.pallas import tpu_sc as plsc

N = 10000
E = 320000
D = 128
H = 8
HID = 8
C = 16

NC = 2
NS = 16
NW = NC * NS

NPAD = 10240
RPW = NPAD // NS
CH = 192
CHUNKS = 54
EPW = CH * CHUNKS
EPAD = EPW * NW

_mesh = plsc.VectorSubcoreMesh(core_axis_name="c", subcore_axis_name="s")


def _iota16():
    return lax.iota(jnp.int32, 16)


def _mp1_body(s_hbm, d_hbm, src3, dst3, zacc,
              acc_out, srcs, dsts, sA, sB, dA, dB, cA, cB,
              acc_sh, gsA, gsB, gdA, gdB, scA, scB):
    c_idx = lax.axis_index("c")
    s_idx = lax.axis_index("s")
    wid = s_idx * NC + c_idx
    r0 = s_idx * RPW
    pltpu.sync_copy(zacc.at[pl.ds(r0, RPW)], acc_sh.at[pl.ds(r0, RPW)])
    pltpu.sync_copy(src3.at[wid], srcs)
    pltpu.sync_copy(dst3.at[wid], dsts)
    plsc.subcore_barrier()

    lanes = _iota16()

    def gstart(j, sbuf, dbuf, gs, gd):
        pltpu.async_copy(s_hbm.at[srcs.at[j]], sbuf, gs)
        pltpu.async_copy(d_hbm.at[dsts.at[j]], dbuf, gd)

    def gwait(sbuf, dbuf, gs, gd):
        pltpu.make_async_copy(s_hbm.at[srcs.at[0]], sbuf, gs).wait()
        pltpu.make_async_copy(d_hbm.at[dsts.at[0]], dbuf, gd).wait()

    def swait(cbuf, sc):
        pltpu.make_async_copy(zacc.at[pl.ds(0, CH)], cbuf, sc).wait()

    def compute(s_rows, d_rows, c_v):
        def blk(i2, carry):
            row = i2 * 16 + lanes
            exs = []
            for k in range(H):
                ck = jnp.full((16,), k, jnp.int32)
                a = plsc.load_gather(s_rows, [row, ck + 64])
                b = plsc.load_gather(d_rows, [row, ck])
                al = a + b
                al = jnp.maximum(al, 0.2 * al)
                e = jnp.exp(al)
                plsc.store_scatter(c_v, [row, ck + 64], e)
                exs.append(e)
            for h in range(H):
                for q in range(HID):
                    col = jnp.full((16,), h * HID + q, jnp.int32)
                    xwv = plsc.load_gather(s_rows, [row, col])
                    plsc.store_scatter(c_v, [row, col], xwv * exs[h])
            return carry

        lax.fori_loop(0, CH // 16, blk, 0)

    gstart(0, sA, dA, gsA, gdA)
    gstart(1, sB, dB, gsB, gdB)

    def pipe(i, carry):
        jA = 2 * i
        jB = 2 * i + 1
        gwait(sA, dA, gsA, gdA)

        @pl.when(i > 0)
        def _():
            swait(cA, scA)

        compute(sA, dA, cA)
        pltpu.async_copy(cA, acc_sh.at[dsts.at[jA]], scA, add=True)

        @pl.when(i < CHUNKS // 2 - 1)
        def _():
            gstart(jA + 2, sA, dA, gsA, gdA)

        gwait(sB, dB, gsB, gdB)

        @pl.when(i > 0)
        def _():
            swait(cB, scB)

        compute(sB, dB, cB)
        pltpu.async_copy(cB, acc_sh.at[dsts.at[jB]], scB, add=True)

        @pl.when(i < CHUNKS // 2 - 1)
        def _():
            gstart(jB + 2, sB, dB, gsB, gdB)

        return carry

    lax.fori_loop(0, CHUNKS // 2, pipe, 0)
    swait(cA, scA)
    swait(cB, scB)
    plsc.subcore_barrier()
    pltpu.sync_copy(acc_sh.at[pl.ds(r0, RPW)], acc_out.at[c_idx, pl.ds(r0, RPW)])


def _mp1(s_tab, d_tab, src3, dst3, zacc):
    kfn = pl.kernel(
        _mp1_body,
        mesh=_mesh,
        compiler_params=pltpu.CompilerParams(
            needs_layout_passes=False, use_tc_tiling_on_sc=False),
        out_type=jax.ShapeDtypeStruct((NC, NPAD, 72), jnp.float32),
        scratch_types=[
            pltpu.VMEM((CHUNKS, CH), jnp.int32),
            pltpu.VMEM((CHUNKS, CH), jnp.int32),
            pltpu.VMEM((CH, 72), jnp.float32),
            pltpu.VMEM((CH, 72), jnp.float32),
            pltpu.VMEM((CH, H), jnp.float32),
            pltpu.VMEM((CH, H), jnp.float32),
            pltpu.VMEM((CH, 72), jnp.float32),
            pltpu.VMEM((CH, 72), jnp.float32),
            pltpu.VMEM_SHARED((NPAD, 72), jnp.float32),
            pltpu.SemaphoreType.DMA,
            pltpu.SemaphoreType.DMA,
            pltpu.SemaphoreType.DMA,
            pltpu.SemaphoreType.DMA,
            pltpu.SemaphoreType.DMA,
            pltpu.SemaphoreType.DMA,
        ],
    )
    return kfn(s_tab, d_tab, src3, dst3, zacc)


W2ACC = 24


def _mp2_body(s_hbm, as_hbm, ad_hbm, src3, dst3, zacc,
              acc_out, srcs, dsts, sA, sB, asA, asB, adA, adB, cA, cB,
              acc_sh, gsA, gsB, gaA, gaB, gdA, gdB, scA, scB):
    c_idx = lax.axis_index("c")
    s_idx = lax.axis_index("s")
    wid = s_idx * NC + c_idx
    r0 = s_idx * RPW
    pltpu.sync_copy(zacc.at[pl.ds(r0, RPW)], acc_sh.at[pl.ds(r0, RPW)])
    pltpu.sync_copy(src3.at[wid], srcs)
    pltpu.sync_copy(dst3.at[wid], dsts)
    plsc.subcore_barrier()

    lanes = _iota16()
    c16 = jnp.full((16,), C, jnp.int32)

    def gstart(j, sbuf, abuf, dbuf, gs, ga, gd):
        pltpu.async_copy(s_hbm.at[srcs.at[j]], sbuf, gs)
        pltpu.async_copy(as_hbm.at[srcs.at[j]], abuf, ga)
        pltpu.async_copy(ad_hbm.at[dsts.at[j]], dbuf, gd)

    def gwait(sbuf, abuf, dbuf, gs, ga, gd):
        pltpu.make_async_copy(s_hbm.at[srcs.at[0]], sbuf, gs).wait()
        pltpu.make_async_copy(as_hbm.at[srcs.at[0]], abuf, ga).wait()
        pltpu.make_async_copy(ad_hbm.at[dsts.at[0]], dbuf, gd).wait()

    def swait(cbuf, sc):
        pltpu.make_async_copy(zacc.at[pl.ds(0, CH)], cbuf, sc).wait()

    def compute(s_rows, as_v, ad_v, c_v):
        def blk(i2, carry):
            row = i2 * 16 + lanes
            sl = pl.ds(i2 * 16, 16)
            al = as_v[sl] + ad_v[sl]
            al = jnp.maximum(al, 0.2 * al)
            e = jnp.exp(al)
            plsc.store_scatter(c_v, [row, c16], e)
            for q in range(C):
                col = jnp.full((16,), q, jnp.int32)
                xwv = plsc.load_gather(s_rows, [row, col])
                plsc.store_scatter(c_v, [row, col], xwv * e)
            return carry

        lax.fori_loop(0, CH // 16, blk, 0)

    gstart(0, sA, asA, adA, gsA, gaA, gdA)
    gstart(1, sB, asB, adB, gsB, gaB, gdB)

    def pipe(i, carry):
        jA = 2 * i
        jB = 2 * i + 1
        gwait(sA, asA, adA, gsA, gaA, gdA)

        @pl.when(i > 0)
        def _():
            swait(cA, scA)

        compute(sA, asA, adA, cA)
        pltpu.async_copy(cA, acc_sh.at[dsts.at[jA]], scA, add=True)

        @pl.when(i < CHUNKS // 2 - 1)
        def _():
            gstart(jA + 2, sA, asA, adA, gsA, gaA, gdA)

        gwait(sB, asB, adB, gsB, gaB, gdB)

        @pl.when(i > 0)
        def _():
            swait(cB, scB)

        compute(sB, asB, adB, cB)
        pltpu.async_copy(cB, acc_sh.at[dsts.at[jB]], scB, add=True)

        @pl.when(i < CHUNKS // 2 - 1)
        def _():
            gstart(jB + 2, sB, asB, adB, gsB, gaB, gdB)

        return carry

    lax.fori_loop(0, CHUNKS // 2, pipe, 0)
    swait(cA, scA)
    swait(cB, scB)
    plsc.subcore_barrier()
    pltpu.sync_copy(acc_sh.at[pl.ds(r0, RPW)], acc_out.at[c_idx, pl.ds(r0, RPW)])


def _mp2(s_tab, as_n, ad_n, src3, dst3, zacc):
    kfn = pl.kernel(
        _mp2_body,
        mesh=_mesh,
        compiler_params=pltpu.CompilerParams(
            needs_layout_passes=False, use_tc_tiling_on_sc=False),
        out_type=jax.ShapeDtypeStruct((NC, NPAD, W2ACC), jnp.float32),
        scratch_types=[
            pltpu.VMEM((CHUNKS, CH), jnp.int32),
            pltpu.VMEM((CHUNKS, CH), jnp.int32),
            pltpu.VMEM((CH, C), jnp.float32),
            pltpu.VMEM((CH, C), jnp.float32),
            pltpu.VMEM((CH,), jnp.float32),
            pltpu.VMEM((CH,), jnp.float32),
            pltpu.VMEM((CH,), jnp.float32),
            pltpu.VMEM((CH,), jnp.float32),
            pltpu.VMEM((CH, W2ACC), jnp.float32),
            pltpu.VMEM((CH, W2ACC), jnp.float32),
            pltpu.VMEM_SHARED((NPAD, W2ACC), jnp.float32),
            pltpu.SemaphoreType.DMA,
            pltpu.SemaphoreType.DMA,
            pltpu.SemaphoreType.DMA,
            pltpu.SemaphoreType.DMA,
            pltpu.SemaphoreType.DMA,
            pltpu.SemaphoreType.DMA,
            pltpu.SemaphoreType.DMA,
            pltpu.SemaphoreType.DMA,
        ],
    )
    return kfn(s_tab, as_n, ad_n, src3, dst3, zacc)


BI = 1024
GRID = NPAD // BI


def _tabs1_body(x_ref, w1_ref, a1s_ref, a1d_ref, s_ref, ad_ref):
    xw = jnp.dot(x_ref[...], w1_ref[...], preferred_element_type=jnp.float32)
    s_ref[:, :H * HID] = xw
    s_ref[:, H * HID:] = jnp.dot(xw, a1s_ref[...],
                                 preferred_element_type=jnp.float32)
    ad_ref[...] = jnp.dot(xw, a1d_ref[...], preferred_element_type=jnp.float32)


def _tc_tabs1(xp, W1, A1s, A1d):
    return pl.pallas_call(
        _tabs1_body,
        grid=(GRID,),
        in_specs=[
            pl.BlockSpec((BI, D), lambda i: (i, 0)),
            pl.BlockSpec((D, H * HID), lambda i: (0, 0)),
            pl.BlockSpec((H * HID, H), lambda i: (0, 0)),
            pl.BlockSpec((H * HID, H), lambda i: (0, 0)),
        ],
        out_specs=[
            pl.BlockSpec((BI, 72), lambda i: (i, 0)),
            pl.BlockSpec((BI, H), lambda i: (i, 0)),
        ],
        out_shape=[
            jax.ShapeDtypeStruct((NPAD, 72), jnp.float32),
            jax.ShapeDtypeStruct((NPAD, H), jnp.float32),
        ],
    )(xp, W1, A1s, A1d)


def _tabs2_body(acc_ref, rexp_ref, b1_ref, w2_ref, a2s_ref, a2d_ref,
                xw2_ref, as2_ref, ad2_ref):
    acct = acc_ref[0] + acc_ref[1]
    numt = acct[:, :H * HID]
    dent = acct[:, H * HID:]
    denx = jnp.dot(dent, rexp_ref[...],
                   preferred_element_type=jnp.float32)
    h = jax.nn.relu(numt / denx + b1_ref[...])
    xw2 = jnp.dot(h, w2_ref[...], preferred_element_type=jnp.float32)
    xw2_ref[...] = xw2
    as2_ref[...] = jnp.dot(xw2, a2s_ref[...], preferred_element_type=jnp.float32)
    ad2_ref[...] = jnp.dot(xw2, a2d_ref[...], preferred_element_type=jnp.float32)


def _tc_tabs2(acc1, Rexp, b1r, W2, a2sT, a2dT):
    return pl.pallas_call(
        _tabs2_body,
        grid=(GRID,),
        in_specs=[
            pl.BlockSpec((NC, BI, 72), lambda i: (0, i, 0)),
            pl.BlockSpec((H, H * HID), lambda i: (0, 0)),
            pl.BlockSpec((1, H * HID), lambda i: (0, 0)),
            pl.BlockSpec((H * HID, C), lambda i: (0, 0)),
            pl.BlockSpec((C, 1), lambda i: (0, 0)),
            pl.BlockSpec((C, 1), lambda i: (0, 0)),
        ],
        out_specs=[
            pl.BlockSpec((BI, C), lambda i: (i, 0)),
            pl.BlockSpec((BI, 1), lambda i: (i, 0)),
            pl.BlockSpec((BI, 1), lambda i: (i, 0)),
        ],
        out_shape=[
            jax.ShapeDtypeStruct((NPAD, C), jnp.float32),
            jax.ShapeDtypeStruct((NPAD, 1), jnp.float32),
            jax.ShapeDtypeStruct((NPAD, 1), jnp.float32),
        ],
    )(acc1, Rexp, b1r, W2, a2sT, a2dT)


def _x1key_body(acc_ref, b2_ref, pw_ref, pb_ref,
                x1_ref, key_ref, gx_ref):
    i = pl.program_id(0)
    acct = acc_ref[0] + acc_ref[1]
    numt = acct[:, :C]
    dent = acct[:, C:C + 1]
    x1 = numt / dent + b2_ref[...]
    x1_ref[...] = x1
    g = jnp.dot(x1, pw_ref[...], preferred_element_type=jnp.float32) \
        + pb_ref[...]
    rows = jax.lax.broadcasted_iota(jnp.int32, (BI, 1), 0) + i * BI
    key_ref[...] = jnp.where(rows < N, g, jnp.inf)
    rows16 = jax.lax.broadcasted_iota(jnp.int32, (BI, C), 0) + i * BI
    gx_ref[...] = jnp.where(rows16 < N, g * x1, 0.0)


def _tc_x1key(acc2, b2r, pwT, pbr):
    return pl.pallas_call(
        _x1key_body,
        grid=(GRID,),
        in_specs=[
            pl.BlockSpec((NC, BI, W2ACC), lambda i: (0, i, 0)),
            pl.BlockSpec((1, C), lambda i: (0, 0)),
            pl.BlockSpec((C, 1), lambda i: (0, 0)),
            pl.BlockSpec((1, 1), lambda i: (0, 0)),
        ],
        out_specs=[
            pl.BlockSpec((BI, C), lambda i: (i, 0)),
            pl.BlockSpec((BI, 1), lambda i: (i, 0)),
            pl.BlockSpec((BI, C), lambda i: (i, 0)),
        ],
        out_shape=[
            jax.ShapeDtypeStruct((NPAD, C), jnp.float32),
            jax.ShapeDtypeStruct((NPAD, 1), jnp.float32),
            jax.ShapeDtypeStruct((NPAD, C), jnp.float32),
        ],
    )(acc2, b2r, pwT, pbr)


BJ = 2048


def _rank_body(ki_ref, kr_ref, rank_ref, acc_ref):
    i = pl.program_id(0)
    ki = ki_ref[...]
    jrel = (jax.lax.broadcasted_iota(jnp.int32, (BI, BJ), 1)
            - jax.lax.broadcasted_iota(jnp.int32, (BI, BJ), 0))
    acc_ref[...] = jnp.zeros((BI, BJ), jnp.float32)

    def jchunk(jc, carry):
        kj = kr_ref[0, pl.ds(jc * BJ, BJ)].reshape(1, BJ)
        lt = kj < ki
        acc_ref[...] += jnp.where(lt, 1.0, 0.0)

        @pl.when(jc * BJ < (i + 1) * BI)
        def _():
            eqb = (kj == ki) & (jrel < i * BI - jc * BJ)
            acc_ref[...] += jnp.where(eqb, 1.0, 0.0)

        return carry

    lax.fori_loop(0, NPAD // BJ, jchunk, 0)
    rank_ref[...] = jnp.sum(acc_ref[...], axis=1, keepdims=True).astype(jnp.int32)


def _tc_rank(key_col, key_row):
    return pl.pallas_call(
        _rank_body,
        grid=(GRID,),
        in_specs=[
            pl.BlockSpec((BI, 1), lambda i: (i, 0)),
            pl.BlockSpec((1, NPAD), lambda i: (0, 0)),
        ],
        out_specs=pl.BlockSpec((BI, 1), lambda i: (i, 0)),
        out_shape=jax.ShapeDtypeStruct((NPAD, 1), jnp.int32),
        scratch_shapes=[pltpu.VMEM((BI, BJ), jnp.float32)],
    )(key_col, key_row)


def _conv_body(sin_ref, wc1_ref, wc2_ref, b1_ref, b2_ref, l2_ref,
               z_ref, h1p_ref):
    acc = jnp.zeros((NPAD, C), jnp.float32)
    for k in range(5):
        acc += jnp.dot(sin_ref[pl.ds(k, NPAD), :], wc1_ref[pl.ds(k * C, C), :],
                       preferred_element_type=jnp.float32)
    rows = jax.lax.broadcasted_iota(jnp.int32, (NPAD, C), 0)
    h1 = jnp.where(rows < N, jax.nn.relu(acc + b1_ref[...]), 0.0)
    h1p_ref[pl.ds(0, 2), :] = jnp.zeros((2, C), jnp.float32)
    h1p_ref[pl.ds(2, NPAD), :] = h1
    h1p_ref[pl.ds(NPAD + 2, 2), :] = jnp.zeros((2, C), jnp.float32)
    acc2 = jnp.zeros((NPAD, C), jnp.float32)
    for k in range(5):
        acc2 += jnp.dot(h1p_ref[pl.ds(k, NPAD), :], wc2_ref[pl.ds(k * C, C), :],
                        preferred_element_type=jnp.float32)
    y = acc2 + b2_ref[...]
    z_ref[...] = jnp.dot(y, l2_ref[...], preferred_element_type=jnp.float32)


def _tc_conv(sin_pad, Wc1, Wc2, c1br, c2br, L2):
    return pl.pallas_call(
        _conv_body,
        in_specs=[
            pl.BlockSpec((NPAD + 4, C), lambda: (0, 0)),
            pl.BlockSpec((5 * C, C), lambda: (0, 0)),
            pl.BlockSpec((5 * C, C), lambda: (0, 0)),
            pl.BlockSpec((1, C), lambda: (0, 0)),
            pl.BlockSpec((1, C), lambda: (0, 0)),
            pl.BlockSpec((C, C), lambda: (0, 0)),
        ],
        out_specs=pl.BlockSpec((NPAD, C), lambda: (0, 0)),
        out_shape=jax.ShapeDtypeStruct((NPAD, C), jnp.float32),
        scratch_shapes=[pltpu.VMEM((NPAD + 4, C), jnp.float32)],
    )(sin_pad, Wc1, Wc2, c1br, c2br, L2)


def _final_body(x1_ref, zu_ref, l1_ref, b_ref, out_ref):
    o = jnp.dot(x1_ref[...], l1_ref[...], preferred_element_type=jnp.float32) \
        + zu_ref[...] + b_ref[...]
    m = jnp.max(o, axis=1, keepdims=True)
    s = o - m
    out_ref[...] = s - jnp.log(jnp.sum(jnp.exp(s), axis=1, keepdims=True))


def _tc_final(x1, zu, L1, br):
    return pl.pallas_call(
        _final_body,
        grid=(GRID,),
        in_specs=[
            pl.BlockSpec((BI, C), lambda i: (i, 0)),
            pl.BlockSpec((BI, C), lambda i: (i, 0)),
            pl.BlockSpec((C, C), lambda i: (0, 0)),
            pl.BlockSpec((1, C), lambda i: (0, 0)),
        ],
        out_specs=pl.BlockSpec((BI, C), lambda i: (i, 0)),
        out_shape=jax.ShapeDtypeStruct((NPAD, C), jnp.float32),
    )(x1, zu, L1, br)


RPW2 = NPAD // NW


def _scat_body(gx_hbm, rank_hbm, out_hbm, idx_v, rows_v, sem1):
    c_idx = lax.axis_index("c")
    s_idx = lax.axis_index("s")
    r0 = (s_idx * NC + c_idx) * RPW2
    pltpu.sync_copy(rank_hbm.at[pl.ds(r0, RPW2)], idx_v)
    pltpu.sync_copy(gx_hbm.at[pl.ds(r0, RPW2)], rows_v)
    pltpu.async_copy(rows_v, out_hbm.at[idx_v], sem1).wait()


def _sc_scatter_rows(gx, rank):
    kfn = pl.kernel(
        _scat_body,
        mesh=_mesh,
        compiler_params=pltpu.CompilerParams(
            needs_layout_passes=False, use_tc_tiling_on_sc=False),
        out_type=jax.ShapeDtypeStruct((NPAD, C), jnp.float32),
        scratch_types=[
            pltpu.VMEM((RPW2,), jnp.int32),
            pltpu.VMEM((RPW2, C), jnp.float32),
            pltpu.SemaphoreType.DMA,
        ],
    )
    return kfn(gx, rank)


def _gath_body(z_hbm, rank_hbm, out_hbm, idx_v, rows_v, sem1):
    c_idx = lax.axis_index("c")
    s_idx = lax.axis_index("s")
    r0 = (s_idx * NC + c_idx) * RPW2
    pltpu.sync_copy(rank_hbm.at[pl.ds(r0, RPW2)], idx_v)
    pltpu.async_copy(z_hbm.at[idx_v], rows_v, sem1).wait()
    pltpu.sync_copy(rows_v, out_hbm.at[pl.ds(r0, RPW2)])


def _sc_gather_rows(z, rank):
    kfn = pl.kernel(
        _gath_body,
        mesh=_mesh,
        compiler_params=pltpu.CompilerParams(
            needs_layout_passes=False, use_tc_tiling_on_sc=False),
        out_type=jax.ShapeDtypeStruct((NPAD, C), jnp.float32),
        scratch_types=[
            pltpu.VMEM((RPW2,), jnp.int32),
            pltpu.VMEM((RPW2, C), jnp.float32),
            pltpu.SemaphoreType.DMA,
        ],
    )
    return kfn(z, rank)


def kernel(x, edge_index, W1, a1_src, a1_dst, b1, W2, a2_src, a2_dst, b2,
           proj_w, proj_b, c1_w, c1_b, c2_w, c2_b, lin_w, lin_b):
    loop = jnp.arange(N, dtype=jnp.int32)
    src = jnp.concatenate([edge_index[0].astype(jnp.int32), loop])
    dst = jnp.concatenate([edge_index[1].astype(jnp.int32), loop])
    npad_e = EPAD - (E + N)
    pad_idx = 10000 + (jnp.arange(npad_e, dtype=jnp.int32) % (NPAD - N))
    srcp = jnp.concatenate([src, pad_idx])
    dstp = jnp.concatenate([dst, pad_idx])

    eyeH = jnp.eye(H, dtype=jnp.float32)
    A1s = (a1_src[:, :, None] * eyeH[:, None, :]).reshape(H * HID, H)
    A1d = (a1_dst[:, :, None] * eyeH[:, None, :]).reshape(H * HID, H)
    Rexp = jnp.kron(eyeH, jnp.ones((1, HID), jnp.float32))
    b1r = b1.reshape(1, H * HID)
    a2sT = a2_src.T
    a2dT = a2_dst.T
    b2r = b2.reshape(1, C)
    pwT = proj_w.T
    pbr = proj_b.reshape(1, 1)
    Wc1 = jnp.transpose(c1_w, (2, 1, 0)).reshape(5 * C, C)
    Wc2 = jnp.transpose(c2_w, (2, 1, 0)).reshape(5 * C, C)
    c1br = c1_b.reshape(1, C)
    c2br = c2_b.reshape(1, C)
    L1 = lin_w[:, :C].T
    L2 = lin_w[:, C:].T
    br = lin_b.reshape(1, C)

    src3 = srcp.reshape(NW, CHUNKS, CH)
    dst3 = dstp.reshape(NW, CHUNKS, CH)

    xp = jnp.pad(x, ((0, NPAD - N), (0, 0)))
    s_tab, ad1 = _tc_tabs1(xp, W1, A1s, A1d)
    zacc1 = jnp.zeros((NPAD, 72), jnp.float32)
    acc1 = _mp1(s_tab, ad1, src3, dst3, zacc1)

    xw2, as2, ad2 = _tc_tabs2(acc1, Rexp, b1r, W2, a2sT, a2dT)
    zacc2 = jnp.zeros((NPAD, W2ACC), jnp.float32)
    acc2 = _mp2(xw2, as2.reshape(NPAD), ad2.reshape(NPAD),
                src3, dst3, zacc2)

    x1p, key, gx = _tc_x1key(acc2, b2r, pwT, pbr)
    rank = _tc_rank(key, key.reshape(1, NPAD)).reshape(NPAD)
    sorted_in = _sc_scatter_rows(gx, rank)

    sin_pad = jnp.pad(sorted_in, ((2, 2), (0, 0)))
    z = _tc_conv(sin_pad, Wc1, Wc2, c1br, c2br, L2)
    zu = _sc_gather_rows(z, rank)
    outp = _tc_final(x1p, zu, L1, br)
    return outp[:N]
